# baseline (speedup 1.0000x reference)
import functools

import jax
import jax.numpy as jnp
from jax import lax
from jax.experimental import pallas as pl
from jax.experimental.pallas import tpu as pltpu
from jax.experimental.pallas import tpu_sc as plsc

_H2 = 0.03 * 0.03
_B = 8
_N = 1024
_LANES = 16
_NWORK = 32
_SC_ROWS = 576
_TC_ROWS = _N - _SC_ROWS
_TC_TILE = 64
_ROWS_PER_W = _B * _SC_ROWS // _NWORK
_RG = 4
_NCHUNK = _N // _LANES


def _sc_body(pts, out, pv, ov):
    cid = lax.axis_index("c")
    sid = lax.axis_index("s")
    wid = sid * 2 + cid
    b = wid // 4
    base = (wid % 4) * _ROWS_PER_W

    pltpu.sync_copy(pts.at[b], pv)

    keep = lax.iota(jnp.int32, _LANES) < 5

    def group(g, acc):
        row0 = base + g * _LANES
        qxv = pv[pl.ds(row0, _LANES)]
        qyv = pv[pl.ds(_N + row0, _LANES)]
        qzv = pv[pl.ds(2 * _N + row0, _LANES)]
        for sub in range(_LANES // _RG):
            qs = []
            for j in range(_RG):
                lane = sub * _RG + j
                qs.append((jnp.full((_LANES,), qxv[lane]),
                           jnp.full((_LANES,), qyv[lane]),
                           jnp.full((_LANES,), qzv[lane])))
            inf16 = jnp.full((_LANES,), jnp.inf, jnp.float32)

            def chunk(c, tops):
                off = c * (2 * _LANES)
                vx1 = pv[pl.ds(off, _LANES)]
                vy1 = pv[pl.ds(_N + off, _LANES)]
                vz1 = pv[pl.ds(2 * _N + off, _LANES)]
                vx2 = pv[pl.ds(off + _LANES, _LANES)]
                vy2 = pv[pl.ds(_N + off + _LANES, _LANES)]
                vz2 = pv[pl.ds(2 * _N + off + _LANES, _LANES)]
                new = []
                for j in range(_RG):
                    qx, qy, qz = qs[j]
                    dx1 = vx1 - qx
                    dy1 = vy1 - qy
                    dz1 = vz1 - qz
                    d1 = dx1 * dx1 + dy1 * dy1 + dz1 * dz1
                    dx2 = vx2 - qx
                    dy2 = vy2 - qy
                    dz2 = vz2 - qz
                    d2 = dx2 * dx2 + dy2 * dy2 + dz2 * dz2
                    s1 = lax.sort(d1, dimension=0, is_stable=False)
                    s2d, _ = plsc.sort_key_val(d2, d2, descending=True)
                    p = jnp.minimum(s1, s2d)
                    pd, _ = plsc.sort_key_val(p, p, descending=True)
                    merged = jnp.minimum(tops[j], pd)
                    new.append(lax.sort(merged, dimension=0, is_stable=False))
                return tuple(new)

            tops = lax.fori_loop(0, _NCHUNK // 2, chunk, (inf16,) * _RG,
                                 unroll=2)
            for j in range(_RG):
                s5 = tops[j]
                contrib = -s5 * jnp.exp(-s5 / _H2)
                acc = acc + jnp.where(keep, contrib, 0.0)
        return acc

    acc = lax.fori_loop(0, _ROWS_PER_W // _LANES, group,
                        jnp.zeros((_LANES,), jnp.float32))
    ov[...] = acc
    pltpu.sync_copy(ov, out.at[wid])


_repulsion_sc = functools.partial(
    pl.kernel,
    mesh=plsc.VectorSubcoreMesh(core_axis_name="c", subcore_axis_name="s"),
    compiler_params=pltpu.CompilerParams(needs_layout_passes=False),
    out_type=jax.ShapeDtypeStruct((_NWORK, _LANES), jnp.float32),
    scratch_types=[
        pltpu.VMEM((3 * _N,), jnp.float32),
        pltpu.VMEM((_LANES,), jnp.float32),
    ],
)(_sc_body)


def _tc_body(qx_ref, qy_ref, qz_ref, px_ref, py_ref, pz_ref, out_ref):
    t = pl.program_id(1)
    qx = qx_ref[0, 0].reshape(_TC_TILE, 1)
    qy = qy_ref[0, 0].reshape(_TC_TILE, 1)
    qz = qz_ref[0, 0].reshape(_TC_TILE, 1)
    px = px_ref[0]
    py = py_ref[0]
    pz = pz_ref[0]
    d = (qx - px) ** 2 + (qy - py) ** 2 + (qz - pz) ** 2
    ci = lax.broadcasted_iota(jnp.int32, (_TC_TILE, _N), 1)
    ri = lax.broadcasted_iota(jnp.int32, (_TC_TILE, _N), 0)
    diag = ci == ri + (_SC_ROWS + t * _TC_TILE)
    d = jnp.where(diag, jnp.inf, d)
    got = jnp.zeros((_TC_TILE, 1), jnp.float32)
    part = jnp.zeros((), jnp.float32)
    for _ in range(4):
        mv = jnp.min(d, axis=1, keepdims=True)
        hit = d == mv
        cnt = jnp.sum(hit.astype(jnp.float32), axis=1, keepdims=True)
        take = jnp.clip(4.0 - got, 0.0, cnt)
        part = part + jnp.sum(take * (-mv * jnp.exp(-mv / _H2)))
        got = got + take
        d = jnp.where(hit, jnp.inf, d)

    @pl.when((pl.program_id(0) == 0) & (pl.program_id(1) == 0))
    def _():
        out_ref[...] = jnp.zeros((1, 1), jnp.float32)

    out_ref[...] += part.reshape(1, 1)


def _tc_q_spec(coord):
    return pl.BlockSpec(
        (1, 1, _TC_TILE),
        lambda b, t, c=coord:
            (b * (3 * _N // _TC_TILE) + (c * _N + _SC_ROWS) // _TC_TILE + t,
             0, 0))


def _tc_p_spec(coord):
    return pl.BlockSpec((1, 1, _N), lambda b, t, c=coord: (b * 3 + c, 0, 0))


_repulsion_tc = pl.pallas_call(
    _tc_body,
    grid=(_B, _TC_ROWS // _TC_TILE),
    in_specs=[_tc_q_spec(0), _tc_q_spec(1), _tc_q_spec(2),
              _tc_p_spec(0), _tc_p_spec(1), _tc_p_spec(2)],
    out_specs=pl.BlockSpec((1, 1), lambda b, t: (0, 0)),
    out_shape=jax.ShapeDtypeStruct((1, 1), jnp.float32),
)


def kernel(pc):
    pts = jnp.transpose(pc, (0, 2, 1)).reshape(_B, 3 * _N)
    partials = _repulsion_sc(pts)
    ptsq = pts.reshape(_B * 3 * _N // _TC_TILE, 1, _TC_TILE)
    ptsp = pts.reshape(_B * 3, 1, _N)
    tc_part = _repulsion_tc(ptsq, ptsq, ptsq, ptsp, ptsp, ptsp)
    return jnp.sum(partials) + tc_part[0, 0]


# speedup vs baseline: 1.3723x; 1.3723x over previous
import functools

import jax
import jax.numpy as jnp
from jax import lax
from jax.experimental import pallas as pl
from jax.experimental.pallas import tpu as pltpu
from jax.experimental.pallas import tpu_sc as plsc

_H2 = 0.03 * 0.03
_B = 8
_N = 1024
_LANES = 16
_NWORK = 32
_SC_ROWS = 640
_TC_ROWS = _N - _SC_ROWS
_TC_TILE = 128
_ROWS_PER_W = _B * _SC_ROWS // _NWORK
_RG = 4
_NCHUNK = _N // _LANES


def _sc_body(pts, out, pv, ov):
    cid = lax.axis_index("c")
    sid = lax.axis_index("s")
    wid = sid * 2 + cid
    b = wid // 4
    base = (wid % 4) * _ROWS_PER_W

    pltpu.sync_copy(pts.at[b], pv)

    keep = lax.iota(jnp.int32, _LANES) < 5

    def group(g, acc):
        row0 = base + g * _LANES
        qxv = pv[pl.ds(row0, _LANES)]
        qyv = pv[pl.ds(_N + row0, _LANES)]
        qzv = pv[pl.ds(2 * _N + row0, _LANES)]
        for sub in range(_LANES // _RG):
            qs = []
            for j in range(_RG):
                lane = sub * _RG + j
                qs.append((jnp.full((_LANES,), qxv[lane]),
                           jnp.full((_LANES,), qyv[lane]),
                           jnp.full((_LANES,), qzv[lane])))
            inf16 = jnp.full((_LANES,), jnp.inf, jnp.float32)

            def chunk(c, tops):
                off = c * (2 * _LANES)
                vx1 = pv[pl.ds(off, _LANES)]
                vy1 = pv[pl.ds(_N + off, _LANES)]
                vz1 = pv[pl.ds(2 * _N + off, _LANES)]
                vx2 = pv[pl.ds(off + _LANES, _LANES)]
                vy2 = pv[pl.ds(_N + off + _LANES, _LANES)]
                vz2 = pv[pl.ds(2 * _N + off + _LANES, _LANES)]
                new = []
                for j in range(_RG):
                    qx, qy, qz = qs[j]
                    dx1 = vx1 - qx
                    dy1 = vy1 - qy
                    dz1 = vz1 - qz
                    d1 = dx1 * dx1 + dy1 * dy1 + dz1 * dz1
                    dx2 = vx2 - qx
                    dy2 = vy2 - qy
                    dz2 = vz2 - qz
                    d2 = dx2 * dx2 + dy2 * dy2 + dz2 * dz2
                    s1 = lax.sort(d1, dimension=0, is_stable=False)
                    s2d, _ = plsc.sort_key_val(d2, d2, descending=True)
                    p = jnp.minimum(s1, s2d)
                    pd, _ = plsc.sort_key_val(p, p, descending=True)
                    merged = jnp.minimum(tops[j], pd)
                    new.append(lax.sort(merged, dimension=0, is_stable=False))
                return tuple(new)

            tops = lax.fori_loop(0, _NCHUNK // 2, chunk, (inf16,) * _RG,
                                 unroll=2)
            for j in range(_RG):
                s5 = tops[j]
                contrib = -s5 * jnp.exp(-s5 / _H2)
                acc = acc + jnp.where(keep, contrib, 0.0)
        return acc

    acc = lax.fori_loop(0, _ROWS_PER_W // _LANES, group,
                        jnp.zeros((_LANES,), jnp.float32))
    ov[...] = acc
    pltpu.sync_copy(ov, out.at[wid])


_repulsion_sc = functools.partial(
    pl.kernel,
    mesh=plsc.VectorSubcoreMesh(core_axis_name="c", subcore_axis_name="s"),
    compiler_params=pltpu.CompilerParams(needs_layout_passes=False),
    out_type=jax.ShapeDtypeStruct((_NWORK, _LANES), jnp.float32),
    scratch_types=[
        pltpu.VMEM((3 * _N,), jnp.float32),
        pltpu.VMEM((_LANES,), jnp.float32),
    ],
)(_sc_body)


def _tc_body(qx_ref, qy_ref, qz_ref, px_ref, py_ref, pz_ref, out_ref):
    t = pl.program_id(1)
    qx = qx_ref[0, 0].reshape(_TC_TILE, 1)
    qy = qy_ref[0, 0].reshape(_TC_TILE, 1)
    qz = qz_ref[0, 0].reshape(_TC_TILE, 1)
    px = px_ref[0]
    py = py_ref[0]
    pz = pz_ref[0]
    d = (qx - px) ** 2 + (qy - py) ** 2 + (qz - pz) ** 2
    ci = lax.broadcasted_iota(jnp.int32, (_TC_TILE, _N), 1)
    ri = lax.broadcasted_iota(jnp.int32, (_TC_TILE, _N), 0)
    diag = ci == ri + (_SC_ROWS + t * _TC_TILE)
    d = jnp.where(diag, jnp.inf, d)
    got = jnp.zeros((_TC_TILE, 1), jnp.float32)
    part = jnp.zeros((), jnp.float32)
    for _ in range(4):
        mv = jnp.min(d, axis=1, keepdims=True)
        hit = d == mv
        cnt = jnp.sum(hit.astype(jnp.float32), axis=1, keepdims=True)
        take = jnp.clip(4.0 - got, 0.0, cnt)
        part = part + jnp.sum(take * (-mv * jnp.exp(-mv / _H2)))
        got = got + take
        d = jnp.where(hit, jnp.inf, d)

    @pl.when((pl.program_id(0) == 0) & (pl.program_id(1) == 0))
    def _():
        out_ref[...] = jnp.zeros((1, 1), jnp.float32)

    out_ref[...] += part.reshape(1, 1)


def _tc_q_spec(coord):
    return pl.BlockSpec(
        (1, 1, _TC_TILE),
        lambda b, t, c=coord:
            (b * (3 * _N // _TC_TILE) + (c * _N + _SC_ROWS) // _TC_TILE + t,
             0, 0))


def _tc_p_spec(coord):
    return pl.BlockSpec((1, 1, _N), lambda b, t, c=coord: (b * 3 + c, 0, 0))


_repulsion_tc = pl.pallas_call(
    _tc_body,
    grid=(_B, _TC_ROWS // _TC_TILE),
    in_specs=[_tc_q_spec(0), _tc_q_spec(1), _tc_q_spec(2),
              _tc_p_spec(0), _tc_p_spec(1), _tc_p_spec(2)],
    out_specs=pl.BlockSpec((1, 1), lambda b, t: (0, 0)),
    out_shape=jax.ShapeDtypeStruct((1, 1), jnp.float32),
)


def kernel(pc):
    pts = jnp.transpose(pc, (0, 2, 1)).reshape(_B, 3 * _N)
    partials = _repulsion_sc(pts)
    ptsq = pts.reshape(_B * 3 * _N // _TC_TILE, 1, _TC_TILE)
    ptsp = pts.reshape(_B * 3, 1, _N)
    tc_part = _repulsion_tc(ptsq, ptsq, ptsq, ptsp, ptsp, ptsp)
    return jnp.sum(partials) + tc_part[0, 0]


# speedup vs baseline: 1.3766x; 1.0032x over previous
import functools

import jax
import jax.numpy as jnp
from jax import lax
from jax.experimental import pallas as pl
from jax.experimental.pallas import tpu as pltpu
from jax.experimental.pallas import tpu_sc as plsc

_H2 = 0.03 * 0.03
_B = 8
_N = 1024
_LANES = 16
_NWORK = 32
_SC_ROWS = 640
_TC_ROWS = _N - _SC_ROWS
_TC_TILE = 128
_ROWS_PER_W = _B * _SC_ROWS // _NWORK
_RG = 4
_NCHUNK = _N // _LANES


def _sc_body(pts, out, pv, ov):
    cid = lax.axis_index("c")
    sid = lax.axis_index("s")
    wid = sid * 2 + cid
    b = wid // 4
    base = (wid % 4) * _ROWS_PER_W

    pltpu.sync_copy(pts.at[b], pv)

    keep = lax.iota(jnp.int32, _LANES) < 5

    def group(g, acc):
        row0 = base + g * _LANES
        qxv = pv[pl.ds(row0, _LANES)]
        qyv = pv[pl.ds(_N + row0, _LANES)]
        qzv = pv[pl.ds(2 * _N + row0, _LANES)]
        for sub in range(_LANES // _RG):
            qs = []
            for j in range(_RG):
                lane = sub * _RG + j
                qs.append((jnp.full((_LANES,), qxv[lane]),
                           jnp.full((_LANES,), qyv[lane]),
                           jnp.full((_LANES,), qzv[lane])))
            inf16 = jnp.full((_LANES,), jnp.inf, jnp.float32)

            def chunk(c, tops):
                off = c * (2 * _LANES)
                vx1 = pv[pl.ds(off, _LANES)]
                vy1 = pv[pl.ds(_N + off, _LANES)]
                vz1 = pv[pl.ds(2 * _N + off, _LANES)]
                vx2 = pv[pl.ds(off + _LANES, _LANES)]
                vy2 = pv[pl.ds(_N + off + _LANES, _LANES)]
                vz2 = pv[pl.ds(2 * _N + off + _LANES, _LANES)]
                new = []
                for j in range(_RG):
                    qx, qy, qz = qs[j]
                    dx1 = vx1 - qx
                    dy1 = vy1 - qy
                    dz1 = vz1 - qz
                    d1 = dx1 * dx1 + dy1 * dy1 + dz1 * dz1
                    dx2 = vx2 - qx
                    dy2 = vy2 - qy
                    dz2 = vz2 - qz
                    d2 = dx2 * dx2 + dy2 * dy2 + dz2 * dz2
                    s1 = lax.sort(d1, dimension=0, is_stable=False)
                    s2d, _ = plsc.sort_key_val(d2, d2, descending=True)
                    p = jnp.minimum(s1, s2d)
                    pd, _ = plsc.sort_key_val(p, p, descending=True)
                    merged = jnp.minimum(tops[j], pd)
                    new.append(lax.sort(merged, dimension=0, is_stable=False))
                return tuple(new)

            tops = lax.fori_loop(0, _NCHUNK // 2, chunk, (inf16,) * _RG,
                                 unroll=2)
            for j in range(_RG):
                s5 = tops[j]
                contrib = -s5 * jnp.exp(-s5 / _H2)
                acc = acc + jnp.where(keep, contrib, 0.0)
        return acc

    acc = lax.fori_loop(0, _ROWS_PER_W // _LANES, group,
                        jnp.zeros((_LANES,), jnp.float32))
    ov[...] = acc
    pltpu.sync_copy(ov, out.at[wid])


_repulsion_sc = functools.partial(
    pl.kernel,
    mesh=plsc.VectorSubcoreMesh(core_axis_name="c", subcore_axis_name="s"),
    compiler_params=pltpu.CompilerParams(needs_layout_passes=False),
    out_type=jax.ShapeDtypeStruct((_NWORK, _LANES), jnp.float32),
    scratch_types=[
        pltpu.VMEM((3 * _N,), jnp.float32),
        pltpu.VMEM((_LANES,), jnp.float32),
    ],
)(_sc_body)


def _tc_body(q3_ref, p3_ref, out_ref):
    t = pl.program_id(1)
    q3 = q3_ref[0]
    p3 = p3_ref[0]
    s = lax.dot_general(q3, p3, (((0,), (0,)), ((), ())),
                        preferred_element_type=jnp.float32)
    qn = jnp.sum(q3 * q3, axis=0, keepdims=True).reshape(_TC_TILE, 1)
    pn = jnp.sum(p3 * p3, axis=0, keepdims=True)
    d = (qn + pn) - (s + s)
    ci = lax.broadcasted_iota(jnp.int32, (_TC_TILE, _N), 1)
    ri = lax.broadcasted_iota(jnp.int32, (_TC_TILE, _N), 0)
    diag = ci == ri + (_SC_ROWS + t * _TC_TILE)
    d = jnp.where(diag, jnp.inf, d)
    got = jnp.zeros((_TC_TILE, 1), jnp.float32)
    part = jnp.zeros((), jnp.float32)
    for _ in range(4):
        mv = jnp.min(d, axis=1, keepdims=True)
        hit = d == mv
        cnt = jnp.sum(hit.astype(jnp.float32), axis=1, keepdims=True)
        take = jnp.clip(4.0 - got, 0.0, cnt)
        part = part + jnp.sum(take * (-mv * jnp.exp(-mv / _H2)))
        got = got + take
        d = jnp.where(hit, jnp.inf, d)

    @pl.when((pl.program_id(0) == 0) & (pl.program_id(1) == 0))
    def _():
        out_ref[...] = jnp.zeros((1, 1), jnp.float32)

    out_ref[...] += part.reshape(1, 1)


_repulsion_tc = pl.pallas_call(
    _tc_body,
    grid=(_B, _TC_ROWS // _TC_TILE),
    in_specs=[
        pl.BlockSpec((1, 3, _TC_TILE), lambda b, t: (b, 0, t)),
        pl.BlockSpec((1, 3, _N), lambda b, t: (b, 0, 0)),
    ],
    out_specs=pl.BlockSpec((1, 1), lambda b, t: (0, 0)),
    out_shape=jax.ShapeDtypeStruct((1, 1), jnp.float32),
)


def kernel(pc):
    pts = jnp.transpose(pc, (0, 2, 1)).reshape(_B, 3 * _N)
    partials = _repulsion_sc(pts)
    p3 = pts.reshape(_B, 3, _N)
    q3 = p3[:, :, _SC_ROWS:]
    tc_part = _repulsion_tc(q3, p3)
    return jnp.sum(partials) + tc_part[0, 0]
